# baseline (speedup 1.0000x reference)
import functools

import jax
import jax.numpy as jnp
from jax import lax
from jax.experimental import pallas as pl
from jax.experimental.pallas import tpu as pltpu
from jax.experimental.pallas import tpu_sc as plsc

DIM = 768
HID = 2048
E = 16
TPAD = 128
ROWS = 16


def _tc_body(len_ref, x_ref, w13_ref, w2_ref, out_ref):
    e = pl.program_id(0)

    def acc(i, s):
        return s + jnp.where(i < e, len_ref[i], 0)
    start = lax.fori_loop(0, E, acc, 0)
    cnt = len_ref[e]

    ri = lax.broadcasted_iota(jnp.int32, (ROWS, TPAD), 0)
    ti = lax.broadcasted_iota(jnp.int32, (ROWS, TPAD), 1)
    sel = jnp.logical_and(ti == start + ri, ri < cnt)
    p = sel.astype(jnp.float32)

    xe = jnp.dot(p, x_ref[...], preferred_element_type=jnp.float32)
    inter = jnp.dot(xe, w13_ref[0], preferred_element_type=jnp.float32)
    x1 = inter[:, :HID]
    x3 = inter[:, HID:]
    h = x1 * jax.nn.sigmoid(x1) * x3
    oe = jnp.dot(h, w2_ref[0], preferred_element_type=jnp.float32)

    @pl.when(e == 0)
    def _():
        out_ref[...] = jnp.zeros_like(out_ref)
    out_ref[...] += jnp.dot(p.T, oe, preferred_element_type=jnp.float32)


def _sc_body(scores_hbm, len_hbm, out_hbm, scores_v, len_v, avg_v, tf_v, ti_v):
    c = lax.axis_index("c")
    s = lax.axis_index("s")

    @pl.when(jnp.logical_and(c == 0, s == 0))
    def _():
        pltpu.sync_copy(scores_hbm, scores_v)
        pltpu.sync_copy(len_hbm, len_v)

        lanes = lax.iota(jnp.int32, 16)

        def cumsum16_f(vec):
            for shift in (1, 2, 4, 8):
                tf_v[...] = vec
                g = plsc.load_gather(tf_v, [jnp.maximum(lanes - shift, 0)])
                vec = vec + jnp.where(lanes >= shift, g, 0.0)
            return vec

        def cumsum16_i(vec):
            for shift in (1, 2, 4, 8):
                ti_v[...] = vec
                g = plsc.load_gather(ti_v, [jnp.maximum(lanes - shift, 0)])
                vec = vec + jnp.where(lanes >= shift, g, 0)
            return vec

        carry = jnp.zeros((16,), jnp.float32)
        for k in range(TPAD // 16):
            cs = cumsum16_f(scores_v[pl.ds(k * 16, 16)])
            scores_v[pl.ds(k * 16, 16)] = cs + carry
            tf_v[...] = cs + carry
            carry = plsc.load_gather(tf_v, [jnp.full((16,), 15, jnp.int32)])

        lv = len_v[...]
        cum = cumsum16_i(lv)
        idx_end = cum - 1
        idx_start = cum - lv - 1
        ge = plsc.load_gather(scores_v, [jnp.maximum(idx_end, 0)])
        gs = plsc.load_gather(scores_v, [jnp.maximum(idx_start, 0)])
        ge = jnp.where(idx_end >= 0, ge, 0.0)
        gs = jnp.where(idx_start >= 0, gs, 0.0)
        denom = jnp.maximum(lv, 1).astype(jnp.float32)
        avg_v[...] = (ge - gs) / denom
        pltpu.sync_copy(avg_v, out_hbm)


_sc_avg = functools.partial(
    pl.kernel,
    out_type=jax.ShapeDtypeStruct((E,), jnp.float32),
    mesh=plsc.VectorSubcoreMesh(core_axis_name="c", subcore_axis_name="s"),
    compiler_params=pltpu.CompilerParams(needs_layout_passes=False),
    scratch_types=[
        pltpu.VMEM((TPAD,), jnp.float32),
        pltpu.VMEM((E,), jnp.int32),
        pltpu.VMEM((E,), jnp.float32),
        pltpu.VMEM((16,), jnp.float32),
        pltpu.VMEM((16,), jnp.int32),
    ],
)(_sc_body)


@jax.jit
def kernel(x, num_tokens_per_expert, top_scores, w13, w2):
    T = x.shape[0]
    xp = jnp.zeros((TPAD, DIM), jnp.float32).at[:T].set(x)
    sp = jnp.zeros((TPAD,), jnp.float32).at[:T].set(top_scores)
    lengths = num_tokens_per_expert.astype(jnp.int32)

    avg = _sc_avg(sp, lengths)

    out_p = pl.pallas_call(
        _tc_body,
        grid=(E,),
        in_specs=[
            pl.BlockSpec(memory_space=pltpu.SMEM),
            pl.BlockSpec((TPAD, DIM), lambda e: (0, 0)),
            pl.BlockSpec((1, DIM, 2 * HID), lambda e: (e, 0, 0)),
            pl.BlockSpec((1, HID, DIM), lambda e: (e, 0, 0)),
        ],
        out_specs=pl.BlockSpec((TPAD, DIM), lambda e: (0, 0)),
        out_shape=jax.ShapeDtypeStruct((TPAD, DIM), jnp.float32),
        compiler_params=pltpu.CompilerParams(
            dimension_semantics=("arbitrary",),
        ),
    )(lengths, xp, w13, w2)

    return out_p[:T], avg
